# baseline (speedup 1.0000x reference)
import functools

import jax
import jax.numpy as jnp
from jax import lax
from jax.experimental import pallas as pl
from jax.experimental.pallas import tpu as pltpu
from jax.experimental.pallas import tpu_sc as plsc

NUM_CHARACTERS = 1000000
EMBED_DIM = 16
BATCH = 16384
NUM_STAGES = 64

_TWLOG = 15
_TW = 1 << _TWLOG
_TGRID = -(-NUM_CHARACTERS // _TW)
_TM = _TW // 8
_TMLOG = _TWLOG - 3
_TBLROWS = _TGRID * _TM


def _transpose_body(i_ref, eye_ref, o_ref):
    x = i_ref[...]
    stacked = jnp.concatenate(
        [x[:, h * _TM:(h + 1) * _TM] for h in range(8)], axis=0)
    o_ref[...] = jax.lax.dot_general(
        stacked, eye_ref[...], (((0,), (0,)), ((), ())),
        preferred_element_type=jnp.float32)


def _transpose(embT, eye):
    return pl.pallas_call(
        _transpose_body,
        grid=(_TGRID,),
        in_specs=[
            pl.BlockSpec((EMBED_DIM, _TW), lambda i: (0, i)),
            pl.BlockSpec((128, 128), lambda i: (0, 0)),
        ],
        out_specs=pl.BlockSpec((_TM, 128), lambda i: (i, 0)),
        out_shape=jax.ShapeDtypeStruct((_TBLROWS, 128), jnp.float32),
    )(embT, eye)


_NC = 2
_NS = 16
_NW = _NC * _NS
_B2 = 2 * BATCH
_BPW = _B2 // _NW
_CHUNK = 128
_STAGE = 256
_ROUNDS = _BPW // _STAGE


@functools.partial(
    pl.kernel,
    out_type=jax.ShapeDtypeStruct((2 * EMBED_DIM, BATCH), jnp.float32),
    mesh=plsc.VectorSubcoreMesh(
        core_axis_name="c", subcore_axis_name="s",
        num_cores=_NC, num_subcores=_NS),
    scratch_types=[
        pltpu.VMEM((_BPW,), jnp.int32),
        pltpu.VMEM((_BPW,), jnp.int32),
        pltpu.VMEM((_BPW,), jnp.int32),
        pltpu.VMEM((_STAGE, 128), jnp.float32),
        pltpu.VMEM((_STAGE, 128), jnp.float32),
        pltpu.VMEM((EMBED_DIM, _STAGE), jnp.float32),
        pltpu.VMEM((EMBED_DIM, _STAGE), jnp.float32),
        pltpu.SemaphoreType.DMA,
        pltpu.SemaphoreType.DMA,
    ],
    compiler_params=pltpu.CompilerParams(needs_layout_passes=False),
)
def _sc_gather(pk_hbm, table_hbm, out_hbm, idx_v, row_v, hcol_v,
               rows_a, rows_b, ext_a, ext_b, sem, wsem):
    wid = lax.axis_index("s") * _NC + lax.axis_index("c")
    base = wid * _BPW
    dbase = EMBED_DIM * (wid // 16)
    cbase = (wid % 16) * _BPW
    pltpu.sync_copy(pk_hbm.at[pl.ds(base, _BPW)], idx_v)
    for j in range(_BPW // 16):
        sl = pl.ds(j * 16, 16)
        pk = idx_v[sl]
        row_v[sl] = jnp.bitwise_and(pk, 0x1FFFF)
        hcol_v[sl] = jnp.left_shift(jnp.right_shift(pk, 17), 4)
    gathers, writes = {}, {}
    lane16 = lax.iota(jnp.int32, 16)

    def fire(r):
        buf = rows_a if r % 2 == 0 else rows_b
        return [pltpu.async_copy(
            table_hbm.at[row_v.at[pl.ds(r * _STAGE + j * _CHUNK, _CHUNK)]],
            buf.at[pl.ds(j * _CHUNK, _CHUNK)], sem)
            for j in range(_STAGE // _CHUNK)]

    def extract_and_write(r):
        for c in gathers[r]:
            c.wait()
        buf = rows_a if r % 2 == 0 else rows_b
        ext = ext_a if r % 2 == 0 else ext_b

        def body(k, _):
            rows16 = lane16 + k * 16
            hcol = hcol_v[pl.ds(r * _STAGE + k * 16, 16)]
            for d in range(EMBED_DIM):
                v = plsc.load_gather(buf, [rows16, hcol + d])
                plsc.store_scatter(
                    ext, [jnp.full((16,), d, jnp.int32), rows16], v)
            return 0

        lax.fori_loop(0, _STAGE // 16, body, 0)
        writes[r] = pltpu.async_copy(
            ext,
            out_hbm.at[pl.ds(dbase, EMBED_DIM),
                       pl.ds(cbase + r * _STAGE, _STAGE)], wsem)

    for r in range(_ROUNDS):
        if r - 2 in writes:
            writes[r - 2].wait()
        gathers[r] = fire(r)
        if r - 1 in gathers:
            extract_and_write(r - 1)
    extract_and_write(_ROUNDS - 1)
    writes[_ROUNDS - 2].wait()
    writes[_ROUNDS - 1].wait()


def _mlp_body(g_ref, w1_ref, b1_ref, w2_ref, b2_ref, o_ref):
    z = jax.lax.dot_general(w1_ref[...], g_ref[...], (((0,), (0,)), ((), ())),
                            preferred_element_type=jnp.float32)
    z = jnp.maximum(z + b1_ref[...], 0.0)
    o_ref[...] = (
        jax.lax.dot_general(w2_ref[...], z, (((0,), (0,)), ((), ())),
                            preferred_element_type=jnp.float32)
        + b2_ref[...])


_MLP_BLK = 4096


def _mlp(g, W1, b1, W2, b2):
    return pl.pallas_call(
        _mlp_body,
        grid=(BATCH // _MLP_BLK,),
        in_specs=[
            pl.BlockSpec((2 * EMBED_DIM, _MLP_BLK), lambda i: (0, i)),
            pl.BlockSpec((2 * EMBED_DIM, 64), lambda i: (0, 0)),
            pl.BlockSpec((64, 1), lambda i: (0, 0)),
            pl.BlockSpec((64, NUM_STAGES), lambda i: (0, 0)),
            pl.BlockSpec((NUM_STAGES, 1), lambda i: (0, 0)),
        ],
        out_specs=pl.BlockSpec((NUM_STAGES, _MLP_BLK), lambda i: (0, i)),
        out_shape=jax.ShapeDtypeStruct((NUM_STAGES, BATCH), jnp.float32),
    )(g, W1, b1.reshape(64, 1), W2, b2.reshape(NUM_STAGES, 1))


def kernel(x, emb, W1, b1, W2, b2):
    x = x.astype(jnp.int32)
    idx = jnp.concatenate([x[:, 0], x[:, 1]])
    rows = ((idx >> _TWLOG) << _TMLOG) + (idx & (_TM - 1))
    lane_grp = (idx >> _TMLOG) & 7
    packed = (lane_grp << 17) | rows
    table = _transpose(emb.T, jnp.eye(128, dtype=jnp.float32))
    g = _sc_gather(packed, table)
    out_t = _mlp(g, W1, b1, W2, b2)
    return out_t.T


# speedup vs baseline: 1.0818x; 1.0818x over previous
import functools

import jax
import jax.numpy as jnp
from jax import lax
from jax.experimental import pallas as pl
from jax.experimental.pallas import tpu as pltpu
from jax.experimental.pallas import tpu_sc as plsc

NUM_CHARACTERS = 1000000
EMBED_DIM = 16
BATCH = 16384
NUM_STAGES = 64

_TWLOG = 16
_TW = 1 << _TWLOG
_TGRID = -(-NUM_CHARACTERS // _TW)
_TM = _TW // 8
_TMLOG = _TWLOG - 3
_TBLROWS = _TGRID * _TM


def _transpose_body(i_ref, eye_ref, o_ref):
    x = i_ref[...]
    stacked = jnp.concatenate(
        [x[:, h * _TM:(h + 1) * _TM] for h in range(8)], axis=0)
    o_ref[...] = jax.lax.dot_general(
        stacked, eye_ref[...], (((0,), (0,)), ((), ())),
        preferred_element_type=jnp.float32)


def _transpose(embT, eye):
    return pl.pallas_call(
        _transpose_body,
        grid=(_TGRID,),
        in_specs=[
            pl.BlockSpec((EMBED_DIM, _TW), lambda i: (0, i)),
            pl.BlockSpec((128, 128), lambda i: (0, 0)),
        ],
        out_specs=pl.BlockSpec((_TM, 128), lambda i: (i, 0)),
        out_shape=jax.ShapeDtypeStruct((_TBLROWS, 128), jnp.float32),
    )(embT, eye)


_NC = 2
_NS = 16
_NW = _NC * _NS
_B2 = 2 * BATCH
_BPW = _B2 // _NW
_CHUNK = 128
_STAGE = 256
_ROUNDS = _BPW // _STAGE


@functools.partial(
    pl.kernel,
    out_type=jax.ShapeDtypeStruct((2 * EMBED_DIM, BATCH), jnp.float32),
    mesh=plsc.VectorSubcoreMesh(
        core_axis_name="c", subcore_axis_name="s",
        num_cores=_NC, num_subcores=_NS),
    scratch_types=[
        pltpu.VMEM((_BPW,), jnp.int32),
        pltpu.VMEM((_BPW,), jnp.int32),
        pltpu.VMEM((_BPW,), jnp.int32),
        pltpu.VMEM((_STAGE, 128), jnp.float32),
        pltpu.VMEM((_STAGE, 128), jnp.float32),
        pltpu.VMEM((EMBED_DIM, _STAGE), jnp.float32),
        pltpu.VMEM((EMBED_DIM, _STAGE), jnp.float32),
        pltpu.SemaphoreType.DMA,
        pltpu.SemaphoreType.DMA,
    ],
    compiler_params=pltpu.CompilerParams(needs_layout_passes=False),
)
def _sc_gather(pk_hbm, table_hbm, out_hbm, idx_v, row_v, hcol_v,
               rows_a, rows_b, ext_a, ext_b, sem, wsem):
    wid = lax.axis_index("s") * _NC + lax.axis_index("c")
    base = wid * _BPW
    dbase = EMBED_DIM * (wid // 16)
    cbase = (wid % 16) * _BPW
    pltpu.sync_copy(pk_hbm.at[pl.ds(base, _BPW)], idx_v)
    for j in range(_BPW // 16):
        sl = pl.ds(j * 16, 16)
        pk = idx_v[sl]
        row_v[sl] = jnp.bitwise_and(pk, 0x1FFFF)
        hcol_v[sl] = jnp.left_shift(jnp.right_shift(pk, 17), 4)
    gathers, writes = {}, {}
    lane16 = lax.iota(jnp.int32, 16)

    def fire(r):
        buf = rows_a if r % 2 == 0 else rows_b
        return [pltpu.async_copy(
            table_hbm.at[row_v.at[pl.ds(r * _STAGE + j * _CHUNK, _CHUNK)]],
            buf.at[pl.ds(j * _CHUNK, _CHUNK)], sem)
            for j in range(_STAGE // _CHUNK)]

    def extract_and_write(r):
        for c in gathers[r]:
            c.wait()
        buf = rows_a if r % 2 == 0 else rows_b
        ext = ext_a if r % 2 == 0 else ext_b

        def body(k, _):
            rows16 = lane16 + k * 16
            hcol = hcol_v[pl.ds(r * _STAGE + k * 16, 16)]
            for d in range(EMBED_DIM):
                v = plsc.load_gather(buf, [rows16, hcol + d])
                plsc.store_scatter(
                    ext, [jnp.full((16,), d, jnp.int32), rows16], v)
            return 0

        lax.fori_loop(0, _STAGE // 16, body, 0)
        writes[r] = pltpu.async_copy(
            ext,
            out_hbm.at[pl.ds(dbase, EMBED_DIM),
                       pl.ds(cbase + r * _STAGE, _STAGE)], wsem)

    for r in range(_ROUNDS):
        if r - 2 in writes:
            writes[r - 2].wait()
        gathers[r] = fire(r)
        if r - 1 in gathers:
            extract_and_write(r - 1)
    extract_and_write(_ROUNDS - 1)
    writes[_ROUNDS - 2].wait()
    writes[_ROUNDS - 1].wait()


def _mlp_body(g_ref, w1_ref, b1_ref, w2_ref, b2_ref, o_ref):
    z = jax.lax.dot_general(w1_ref[...], g_ref[...], (((0,), (0,)), ((), ())),
                            preferred_element_type=jnp.float32)
    z = jnp.maximum(z + b1_ref[...], 0.0)
    o_ref[...] = (
        jax.lax.dot_general(w2_ref[...], z, (((0,), (0,)), ((), ())),
                            preferred_element_type=jnp.float32)
        + b2_ref[...])


_MLP_BLK = 4096


def _mlp(g, W1, b1, W2, b2):
    return pl.pallas_call(
        _mlp_body,
        grid=(BATCH // _MLP_BLK,),
        in_specs=[
            pl.BlockSpec((2 * EMBED_DIM, _MLP_BLK), lambda i: (0, i)),
            pl.BlockSpec((2 * EMBED_DIM, 64), lambda i: (0, 0)),
            pl.BlockSpec((64, 1), lambda i: (0, 0)),
            pl.BlockSpec((64, NUM_STAGES), lambda i: (0, 0)),
            pl.BlockSpec((NUM_STAGES, 1), lambda i: (0, 0)),
        ],
        out_specs=pl.BlockSpec((NUM_STAGES, _MLP_BLK), lambda i: (0, i)),
        out_shape=jax.ShapeDtypeStruct((NUM_STAGES, BATCH), jnp.float32),
    )(g, W1, b1.reshape(64, 1), W2, b2.reshape(NUM_STAGES, 1))


def kernel(x, emb, W1, b1, W2, b2):
    x = x.astype(jnp.int32)
    idx = jnp.concatenate([x[:, 0], x[:, 1]])
    rows = ((idx >> _TWLOG) << _TMLOG) + (idx & (_TM - 1))
    lane_grp = (idx >> _TMLOG) & 7
    packed = (lane_grp << 17) | rows
    table = _transpose(emb.T, jnp.eye(128, dtype=jnp.float32))
    g = _sc_gather(packed, table)
    out_t = _mlp(g, W1, b1, W2, b2)
    return out_t.T
